# baseline (speedup 1.0000x reference)
import functools

import jax
import jax.numpy as jnp
from jax import lax
from jax.experimental import pallas as pl
from jax.experimental.pallas import tpu as pltpu
from jax.experimental.pallas import tpu_sc as plsc


def _proj_body(x_ref, w_ref, wb_ref, p_ref):
    p = jnp.dot(w_ref[...], x_ref[0], preferred_element_type=jnp.float32)
    p_ref[0] = jnp.exp(-(p + wb_ref[...]))


def _make_sc_pairwise(B, N):
    L = 16
    NW = 32
    RPW = (B * N) // NW
    G = 16
    NG = RPW // G
    CH = N // L
    half_chunks = RPW // L

    mesh = plsc.VectorSubcoreMesh(core_axis_name="c", subcore_axis_name="s")

    @functools.partial(
        pl.kernel,
        out_type=jax.ShapeDtypeStruct((B * N, N), jnp.float32),
        mesh=mesh,
        scratch_types=[
            pltpu.VMEM((N,), jnp.float32),
            pltpu.VMEM((N,), jnp.float32),
            pltpu.VMEM((G, N), jnp.float32),
            pltpu.VMEM((G, N), jnp.float32),
            pltpu.SemaphoreType.DMA,
            pltpu.SemaphoreType.DMA,
        ],
    )
    def sc_pair(u_hbm, v_hbm, out_hbm, u_v, v_v, buf0, buf1, sem0, sem1):
        cid = lax.axis_index("c")
        sid = lax.axis_index("s")
        wid = sid * 2 + cid
        bw = wid // 2
        half = wid % 2
        base_row = wid * RPW
        cg0 = half * half_chunks

        pltpu.sync_copy(u_hbm.at[bw], u_v)
        pltpu.sync_copy(v_hbm.at[bw], v_v)

        jl = lax.iota(jnp.int32, 16)

        def emit_group(g, buf):
            cg = cg0 + g
            uc = u_v[pl.ds(cg * L, L)]
            vc = v_v[pl.ds(cg * L, L)]
            for r in range(G):
                u_i = uc[r]
                v_i = vc[r]

                @plsc.parallel_loop(0, cg, unroll=4)
                def _lower(c):
                    buf[r, pl.ds(c * L, L)] = 1.0 / (
                        1.0 + u_v[pl.ds(c * L, L)] * v_i)

                @plsc.parallel_loop(cg + 1, CH, unroll=4)
                def _upper(c):
                    buf[r, pl.ds(c * L, L)] = 1.0 / (
                        1.0 + u_i * v_v[pl.ds(c * L, L)])

                e = jnp.where(jl > r, u_i * vc, uc * v_i)
                s = 1.0 / (1.0 + e)
                buf[r, pl.ds(cg * L, L)] = jnp.where(jl == r, 0.0, s)

        def loop_body(gg, _):
            g0 = gg * 2
            g1 = gg * 2 + 1

            @pl.when(gg > 0)
            def _():
                pltpu.make_async_copy(
                    buf0, out_hbm.at[pl.ds(base_row, G)], sem0).wait()

            emit_group(g0, buf0)
            pltpu.async_copy(
                buf0, out_hbm.at[pl.ds(base_row + g0 * G, G)], sem0)

            @pl.when(gg > 0)
            def _():
                pltpu.make_async_copy(
                    buf1, out_hbm.at[pl.ds(base_row, G)], sem1).wait()

            emit_group(g1, buf1)
            pltpu.async_copy(
                buf1, out_hbm.at[pl.ds(base_row + g1 * G, G)], sem1)
            return 0

        lax.fori_loop(0, NG // 2, loop_body, 0)
        pltpu.make_async_copy(buf0, out_hbm.at[pl.ds(base_row, G)], sem0).wait()
        pltpu.make_async_copy(buf1, out_hbm.at[pl.ds(base_row, G)], sem1).wait()

    return sc_pair


def kernel(node_features, W, b):
    B, F, N = node_features.shape
    Wr = W.reshape(2, F)
    Wb = jnp.stack([jnp.zeros((), W.dtype), b[0]]).reshape(2, 1)

    uv = pl.pallas_call(
        _proj_body,
        grid=(B,),
        in_specs=[
            pl.BlockSpec((1, F, N), lambda i: (i, 0, 0)),
            pl.BlockSpec((2, F), lambda i: (0, 0)),
            pl.BlockSpec((2, 1), lambda i: (0, 0)),
        ],
        out_specs=pl.BlockSpec((1, 2, N), lambda i: (i, 0, 0)),
        out_shape=jax.ShapeDtypeStruct((B, 2, N), jnp.float32),
    )(node_features, Wr, Wb)
    u = uv[:, 0, :]
    v = uv[:, 1, :]

    out_flat = _make_sc_pairwise(B, N)(u, v)
    return out_flat.reshape(B, N, N)


# speedup vs baseline: 3.5152x; 3.5152x over previous
import functools

import jax
import jax.numpy as jnp
from jax import lax
from jax.experimental import pallas as pl
from jax.experimental.pallas import tpu as pltpu
from jax.experimental.pallas import tpu_sc as plsc


def _proj_body(x_ref, w_ref, wb_ref, p_ref):
    p = jnp.dot(w_ref[...], x_ref[0], preferred_element_type=jnp.float32)
    p_ref[0] = jnp.exp(-(p + wb_ref[...]))


def _make_sc_pairwise(B, N):
    L = 16
    NW = 32
    RPW = (B * N) // NW
    G = 16
    NG = RPW // G
    CH = N // L
    half_chunks = RPW // L

    mesh = plsc.VectorSubcoreMesh(core_axis_name="c", subcore_axis_name="s")

    @functools.partial(
        pl.kernel,
        out_type=jax.ShapeDtypeStruct((B * N, N), jnp.float32),
        mesh=mesh,
        scratch_types=[
            pltpu.VMEM((N,), jnp.float32),
            pltpu.VMEM((N,), jnp.float32),
            pltpu.VMEM((G, N), jnp.float32),
            pltpu.VMEM((G, N), jnp.float32),
            pltpu.SemaphoreType.DMA,
            pltpu.SemaphoreType.DMA,
        ],
    )
    def sc_pair(u_hbm, v_hbm, out_hbm, u_v, v_v, buf0, buf1, sem0, sem1):
        cid = lax.axis_index("c")
        sid = lax.axis_index("s")
        wid = sid * 2 + cid
        bw = wid // 2
        half = wid % 2
        base_row = wid * RPW
        cg0 = half * half_chunks

        pltpu.sync_copy(u_hbm.at[bw], u_v)
        pltpu.sync_copy(v_hbm.at[bw], v_v)

        jl = lax.iota(jnp.int32, 16)

        def emit_group(g, buf):
            cg = cg0 + g
            uc = u_v[pl.ds(cg * L, L)]
            vc = v_v[pl.ds(cg * L, L)]
            u_spl = [jnp.full((L,), uc[r], dtype=jnp.float32) for r in range(G)]
            v_spl = [jnp.full((L,), vc[r], dtype=jnp.float32) for r in range(G)]

            @plsc.parallel_loop(0, cg, unroll=2)
            def _lower(c):
                uch = u_v[pl.ds(c * L, L)]
                for r in range(G):
                    buf[r, pl.ds(c * L, L)] = 1.0 / (1.0 + uch * v_spl[r])

            @plsc.parallel_loop(cg + 1, CH, unroll=2)
            def _upper(c):
                vch = v_v[pl.ds(c * L, L)]
                for r in range(G):
                    buf[r, pl.ds(c * L, L)] = 1.0 / (1.0 + u_spl[r] * vch)

            for r in range(G):
                e = jnp.where(jl > r, u_spl[r] * vc, uc * v_spl[r])
                s = 1.0 / (1.0 + e)
                buf[r, pl.ds(cg * L, L)] = jnp.where(jl == r, 0.0, s)

        def loop_body(gg, _):
            g0 = gg * 2
            g1 = gg * 2 + 1

            @pl.when(gg > 0)
            def _():
                pltpu.make_async_copy(
                    buf0, out_hbm.at[pl.ds(base_row, G)], sem0).wait()

            emit_group(g0, buf0)
            pltpu.async_copy(
                buf0, out_hbm.at[pl.ds(base_row + g0 * G, G)], sem0)

            @pl.when(gg > 0)
            def _():
                pltpu.make_async_copy(
                    buf1, out_hbm.at[pl.ds(base_row, G)], sem1).wait()

            emit_group(g1, buf1)
            pltpu.async_copy(
                buf1, out_hbm.at[pl.ds(base_row + g1 * G, G)], sem1)
            return 0

        lax.fori_loop(0, NG // 2, loop_body, 0)
        pltpu.make_async_copy(buf0, out_hbm.at[pl.ds(base_row, G)], sem0).wait()
        pltpu.make_async_copy(buf1, out_hbm.at[pl.ds(base_row, G)], sem1).wait()

    return sc_pair


def kernel(node_features, W, b):
    B, F, N = node_features.shape
    Wr = W.reshape(2, F)
    Wb = jnp.stack([jnp.zeros((), W.dtype), b[0]]).reshape(2, 1)

    uv = pl.pallas_call(
        _proj_body,
        grid=(B,),
        in_specs=[
            pl.BlockSpec((1, F, N), lambda i: (i, 0, 0)),
            pl.BlockSpec((2, F), lambda i: (0, 0)),
            pl.BlockSpec((2, 1), lambda i: (0, 0)),
        ],
        out_specs=pl.BlockSpec((1, 2, N), lambda i: (i, 0, 0)),
        out_shape=jax.ShapeDtypeStruct((B, 2, N), jnp.float32),
    )(node_features, Wr, Wb)
    u = uv[:, 0, :]
    v = uv[:, 1, :]

    out_flat = _make_sc_pairwise(B, N)(u, v)
    return out_flat.reshape(B, N, N)
